# baseline (speedup 1.0000x reference)
import functools

import jax
import jax.numpy as jnp
from jax.experimental import pallas as pl
from jax.experimental.pallas import tpu as pltpu

_TILE_S = 384


def _router_kernel(x_ref, w_ref, o_ref):
    b = x_ref.shape[0]
    w = w_ref[...]
    for i in range(b):
        logits = jax.lax.dot_general(
            w, x_ref[i],
            dimension_numbers=(((1,), (1,)), ((), ())),
            preferred_element_type=jnp.float32,
        )
        m = jnp.max(logits, axis=0, keepdims=True)
        ex = jnp.exp(logits - m)
        o_ref[i] = ex / jnp.sum(ex, axis=0, keepdims=True)


@functools.partial(jax.jit, static_argnames=("interpret",))
def kernel(x, expert_embeddings, interpret=False):
    B, S, H = x.shape
    E = expert_embeddings.shape[0]
    out = pl.pallas_call(
        _router_kernel,
        grid=(pl.cdiv(S, _TILE_S),),
        in_specs=[
            pl.BlockSpec((B, _TILE_S, H), lambda i: (0, i, 0)),
            pl.BlockSpec((E, H), lambda i: (0, 0)),
        ],
        out_specs=pl.BlockSpec((B, E, _TILE_S), lambda i: (0, 0, i)),
        out_shape=jax.ShapeDtypeStruct((B, E, S), jnp.float32),
        compiler_params=pltpu.CompilerParams(
            dimension_semantics=("arbitrary",),
            vmem_limit_bytes=60 * 1024 * 1024,
        ),
        interpret=interpret,
    )(x, expert_embeddings)
    return jnp.transpose(out, (0, 2, 1))


# speedup vs baseline: 1.0356x; 1.0356x over previous
import jax
import jax.numpy as jnp
from jax.experimental import pallas as pl
from jax.experimental.pallas import tpu as pltpu

_TILE_S = 256


def _router_kernel(x_ref, w_ref, o_ref):
    b = x_ref.shape[0]
    w = w_ref[...]
    for i in range(b):
        logits = jax.lax.dot_general(
            w, x_ref[i],
            dimension_numbers=(((1,), (1,)), ((), ())),
            preferred_element_type=jnp.float32,
        )
        m = jnp.max(logits, axis=0, keepdims=True)
        ex = jnp.exp(logits - m)
        o_ref[i] = ex / jnp.sum(ex, axis=0, keepdims=True)


@jax.jit
def kernel(x, expert_embeddings):
    B, S, H = x.shape
    E = expert_embeddings.shape[0]
    out = pl.pallas_call(
        _router_kernel,
        grid=(S // _TILE_S,),
        in_specs=[
            pl.BlockSpec((B, _TILE_S, H), lambda i: (0, i, 0)),
            pl.BlockSpec((E, H), lambda i: (0, 0)),
        ],
        out_specs=pl.BlockSpec((B, E, _TILE_S), lambda i: (0, 0, i)),
        out_shape=jax.ShapeDtypeStruct((B, E, S), jnp.float32),
        compiler_params=pltpu.CompilerParams(
            dimension_semantics=("arbitrary",),
        ),
    )(x, expert_embeddings)
    return jnp.transpose(out, (0, 2, 1))
